# baseline (speedup 1.0000x reference)
import jax
import jax.numpy as jnp
from jax.experimental import pallas as pl
from jax.experimental.pallas import tpu as pltpu

_N = 10000
_BI = 400
_NB = _N // _BI


def _gcn_body(adj_ref, x_ref, w1_ref, w2_ref, z_ref, h0_ref, h_ref, y_ref):
    p = pl.program_id(0)
    i = pl.program_id(1)

    @pl.when((p == 0) & (i == 0))
    def _():
        h0_ref[...] = jnp.dot(x_ref[...], w1_ref[...],
                              preferred_element_type=jnp.float32)

    @pl.when(p == 0)
    def _():
        h = jnp.dot(adj_ref[...], h0_ref[...],
                    preferred_element_type=jnp.float32)
        h_ref[pl.ds(i * _BI, _BI), :] = jnp.maximum(h, 0.0)

    @pl.when((p == 1) & (i == 0))
    def _():
        y_ref[...] = jnp.dot(h_ref[...], w2_ref[...],
                             preferred_element_type=jnp.float32)

    @pl.when(p == 1)
    def _():
        z_ref[...] = jnp.dot(adj_ref[...], y_ref[...],
                             preferred_element_type=jnp.float32)


def kernel(adj, x, W1, W2):
    n, d_in = x.shape
    h1 = W1.shape[1]
    h2 = W2.shape[1]
    return pl.pallas_call(
        _gcn_body,
        grid=(2, _NB),
        in_specs=[
            pl.BlockSpec((_BI, n), lambda p, i: (i, 0)),
            pl.BlockSpec((n, d_in), lambda p, i: (0, 0)),
            pl.BlockSpec((d_in, h1), lambda p, i: (0, 0)),
            pl.BlockSpec((h1, h2), lambda p, i: (0, 0)),
        ],
        out_specs=pl.BlockSpec((_BI, h2), lambda p, i: (i, 0)),
        out_shape=jax.ShapeDtypeStruct((n, h2), jnp.float32),
        scratch_shapes=[
            pltpu.VMEM((n, h1), jnp.float32),
            pltpu.VMEM((n, h1), jnp.float32),
            pltpu.VMEM((n, h2), jnp.float32),
        ],
    )(adj, x, W1, W2)


# speedup vs baseline: 1.0039x; 1.0039x over previous
import jax
import jax.numpy as jnp
from jax.experimental import pallas as pl
from jax.experimental.pallas import tpu as pltpu

_N = 10000
_BI = 400
_NB = _N // _BI


def _gcn_body(adj_ref, x_ref, w1_ref, w2_ref, z_ref, h0_ref, y_ref):
    p = pl.program_id(0)
    i = pl.program_id(1)

    @pl.when((p == 0) & (i == 0))
    def _():
        h0_ref[...] = jnp.dot(x_ref[...], w1_ref[...],
                              preferred_element_type=jnp.float32)

    @pl.when(p == 0)
    def _():
        h = jnp.dot(adj_ref[...], h0_ref[...],
                    preferred_element_type=jnp.float32)
        y_ref[pl.ds(i * _BI, _BI), :] = jnp.dot(
            jnp.maximum(h, 0.0), w2_ref[...],
            preferred_element_type=jnp.float32)

    @pl.when(p == 1)
    def _():
        z_ref[...] = jnp.dot(adj_ref[...], y_ref[...],
                             preferred_element_type=jnp.float32)


def kernel(adj, x, W1, W2):
    n, d_in = x.shape
    h1 = W1.shape[1]
    h2 = W2.shape[1]
    return pl.pallas_call(
        _gcn_body,
        grid=(2, _NB),
        in_specs=[
            pl.BlockSpec((_BI, n), lambda p, i: (i, 0)),
            pl.BlockSpec((n, d_in), lambda p, i: (0, 0)),
            pl.BlockSpec((d_in, h1), lambda p, i: (0, 0)),
            pl.BlockSpec((h1, h2), lambda p, i: (0, 0)),
        ],
        out_specs=pl.BlockSpec((_BI, h2), lambda p, i: (i, 0)),
        out_shape=jax.ShapeDtypeStruct((n, h2), jnp.float32),
        scratch_shapes=[
            pltpu.VMEM((n, h1), jnp.float32),
            pltpu.VMEM((n, h2), jnp.float32),
        ],
    )(adj, x, W1, W2)
